# Initial kernel scaffold; baseline (speedup 1.0000x reference)
import jax
import jax.numpy as jnp
from jax.experimental import pallas as pl


def kernel(x, edge_index, Wh, att_src, att_dst, bias_h, W_fc, b_fc, ln_gamma, ln_beta, W_g, b_g):
    raise NotImplementedError("write your pallas kernel here")


# speedup vs baseline: 42.4010x; 42.4010x over previous
import functools

import jax
import jax.numpy as jnp
from jax import lax
from jax.experimental import pallas as pl
from jax.experimental.pallas import tpu as pltpu
from jax.experimental.pallas import tpu_sc as plsc

N, E, D, H, HD = 10000, 320000, 128, 8, 16

NC, NS = 2, 16
NW = NC * NS
EPW = E // NW
K = 80
NCH = EPW // K
NP = 10240
RPT = NP // NS


def _sc_edges(src_hbm, dst_hbm, xp_hbm, es_hbm, ed_hbm, z128_hbm,
              acc_out, den_out,
              acc_sh, sidx, didx, xp_rows, es_rows, ed_rows, wexp, sem):
    c = lax.axis_index("c")
    s = lax.axis_index("s")
    wid = c * NS + s
    rb = s * RPT

    pltpu.sync_copy(z128_hbm.at[pl.ds(0, K)], wexp)

    def zero_acc(j, carry):
        pltpu.sync_copy(wexp, acc_sh.at[pl.ds(rb + j * K, K)])
        return carry

    def copy_out(out_ref):
        def body(j, carry):
            pltpu.sync_copy(acc_sh.at[pl.ds(rb + j * K, K)], xp_rows)
            pltpu.sync_copy(xp_rows, out_ref.at[pl.ds(c * NP + rb + j * K, K)])
            return carry

        lax.fori_loop(0, RPT // K, body, 0)

    def make_chunk(do_msg):
        def chunk(i, carry):
            base = wid * EPW + i * K
            pltpu.sync_copy(src_hbm.at[pl.ds(base, K)], sidx)
            pltpu.sync_copy(dst_hbm.at[pl.ds(base, K)], didx)
            g1 = pltpu.async_copy(es_hbm.at[sidx], es_rows, sem)
            g2 = pltpu.async_copy(ed_hbm.at[didx], ed_rows, sem)
            g3 = pltpu.async_copy(xp_hbm.at[sidx], xp_rows, sem) if do_msg else None
            g1.wait()
            g2.wait()
            if do_msg:
                g3.wait()

            def eloop(e, carry2):
                v = es_rows[e, pl.ds(0, 16)] + ed_rows[e, pl.ds(0, 16)]
                v = jnp.where(v >= 0.0, v, 0.2 * v)
                w = jnp.exp(v)
                for h in range(H):
                    if do_msg:
                        xv = xp_rows[e, pl.ds(h * HD, HD)]
                        xp_rows[e, pl.ds(h * HD, HD)] = xv * w[h]
                    else:
                        wexp[e, pl.ds(h * HD, HD)] = jnp.broadcast_to(w[h], (HD,))
                return carry2

            lax.fori_loop(0, K, eloop, 0)
            pltpu.sync_copy(xp_rows if do_msg else wexp, acc_sh.at[didx], add=True)
            return carry

        return chunk

    lax.fori_loop(0, RPT // K, zero_acc, 0)
    plsc.subcore_barrier()
    lax.fori_loop(0, NCH, make_chunk(True), 0)
    plsc.subcore_barrier()
    copy_out(acc_out)
    plsc.subcore_barrier()
    lax.fori_loop(0, RPT // K, zero_acc, 0)
    plsc.subcore_barrier()
    lax.fori_loop(0, NCH, make_chunk(False), 0)
    plsc.subcore_barrier()
    copy_out(den_out)


def _make_sc_call():
    mesh = plsc.VectorSubcoreMesh(core_axis_name="c", subcore_axis_name="s")
    return pl.kernel(
        _sc_edges,
        out_type=[
            jax.ShapeDtypeStruct((2 * NP, D), jnp.float32),
            jax.ShapeDtypeStruct((2 * NP, D), jnp.float32),
        ],
        mesh=mesh,
        scratch_types=[
            pltpu.VMEM_SHARED((NP, D), jnp.float32),
            pltpu.VMEM((K,), jnp.int32),
            pltpu.VMEM((K,), jnp.int32),
            pltpu.VMEM((K, D), jnp.float32),
            pltpu.VMEM((K, D), jnp.float32),
            pltpu.VMEM((K, D), jnp.float32),
            pltpu.VMEM((K, D), jnp.float32),
            pltpu.SemaphoreType.DMA,
        ],
    )


B = 1000
G = N // B


def _tc_pre(x_ref, wall_ref, as_ref, ad_ref, xp_ref, es_ref, ed_ref):
    xp = jnp.dot(x_ref[...], wall_ref[...], preferred_element_type=jnp.float32)
    xp_ref[...] = xp
    es_ref[...] = jnp.dot(xp, as_ref[...], preferred_element_type=jnp.float32)
    ed_ref[...] = jnp.dot(xp, ad_ref[...], preferred_element_type=jnp.float32)


def _tc_mid(a0_ref, a1_ref, d0_ref, d1_ref, bias_ref, wfct_ref,
            bfc_ref, gam_ref, bet_ref, xl_ref, xs_ref):
    acc = a0_ref[...] + a1_ref[...]
    den = d0_ref[...] + d1_ref[...]
    x_local = acc / (den + 1e-16) + bias_ref[...]

    sa = jnp.dot(x_local, wfct_ref[...],
                 preferred_element_type=jnp.float32) + bfc_ref[...]
    sa = jnp.where(sa >= 0.0, sa, 0.01 * sa)
    m = jnp.max(sa, axis=1, keepdims=True)
    p = jnp.exp(sa - m)
    sa = p / jnp.sum(p, axis=1, keepdims=True)

    xl = x_local * sa
    xl = jnp.where(xl >= 0.0, xl, 0.2 * xl)
    xl = jnp.dot(xl, wfct_ref[...],
                 preferred_element_type=jnp.float32) + bfc_ref[...]

    mu = jnp.mean(xl, axis=1, keepdims=True)
    xc = xl - mu
    var = jnp.mean(xc * xc, axis=1, keepdims=True)
    xl = xc * lax.rsqrt(var + 1e-5) * gam_ref[...] + bet_ref[...]

    nrm = jnp.sqrt(jnp.sum(xl * xl, axis=1, keepdims=True))
    xl = xl / jnp.maximum(nrm, 1e-12)

    xl_ref[...] = xl
    part = jnp.broadcast_to(jnp.sum(xl, axis=0, keepdims=True), (8, D))

    @pl.when(pl.program_id(0) == 0)
    def _():
        xs_ref[...] = part

    @pl.when(pl.program_id(0) != 0)
    def _():
        xs_ref[...] = xs_ref[...] + part


def _tc_fin(xs_ref, wgt_ref, bg_ref, xl_ref, out_ref):
    xg = jnp.sum(xs_ref[...], axis=0, keepdims=True) * (1.0 / (8.0 * N))
    g = jnp.dot(xg, wgt_ref[...], preferred_element_type=jnp.float32) + bg_ref[...]
    g = jnp.maximum(g, 0.0)
    m = jnp.max(g, axis=1, keepdims=True)
    p = jnp.exp(g - m)
    ga = p / jnp.sum(p, axis=1, keepdims=True)
    out_ref[...] = xl_ref[...] * ga


def kernel(x, edge_index, Wh, att_src, att_dst, bias_h, W_fc, b_fc,
           ln_gamma, ln_beta, W_g, b_g):
    f32 = jnp.float32
    W_all = jnp.transpose(Wh, (1, 0, 2)).reshape(D, H * HD)
    KR = jnp.kron(jnp.eye(H, dtype=f32), jnp.ones((HD, 1), dtype=f32))
    padw = jnp.zeros((H * HD, 120), f32)
    A_s = jnp.concatenate([att_src.reshape(H * HD)[:, None] * KR, padw], 1)
    A_d = jnp.concatenate([att_dst.reshape(H * HD)[:, None] * KR, padw], 1)
    bias_flat = bias_h.reshape(1, H * HD)
    src = edge_index[0]
    dst = edge_index[1]

    xp, es, ed = pl.pallas_call(
        _tc_pre,
        grid=(G,),
        in_specs=[
            pl.BlockSpec((B, D), lambda i: (i, 0)),
            pl.BlockSpec((D, D), lambda i: (0, 0)),
            pl.BlockSpec((D, D), lambda i: (0, 0)),
            pl.BlockSpec((D, D), lambda i: (0, 0)),
        ],
        out_specs=[
            pl.BlockSpec((B, D), lambda i: (i, 0)),
            pl.BlockSpec((B, D), lambda i: (i, 0)),
            pl.BlockSpec((B, D), lambda i: (i, 0)),
        ],
        out_shape=[
            jax.ShapeDtypeStruct((N, D), f32),
            jax.ShapeDtypeStruct((N, D), f32),
            jax.ShapeDtypeStruct((N, D), f32),
        ],
    )(x, W_all, A_s, A_d)

    z128 = jnp.zeros((NP, D), f32)
    acc2, den2 = _make_sc_call()(src, dst, xp, es, ed, z128)

    xl, xs = pl.pallas_call(
        _tc_mid,
        grid=(G,),
        in_specs=[
            pl.BlockSpec((B, D), lambda i: (i, 0)),
            pl.BlockSpec((B, D), lambda i: (i, 0)),
            pl.BlockSpec((B, D), lambda i: (i, 0)),
            pl.BlockSpec((B, D), lambda i: (i, 0)),
            pl.BlockSpec((1, D), lambda i: (0, 0)),
            pl.BlockSpec((D, D), lambda i: (0, 0)),
            pl.BlockSpec((1, D), lambda i: (0, 0)),
            pl.BlockSpec((1, D), lambda i: (0, 0)),
            pl.BlockSpec((1, D), lambda i: (0, 0)),
        ],
        out_specs=[
            pl.BlockSpec((B, D), lambda i: (i, 0)),
            pl.BlockSpec((8, D), lambda i: (0, 0)),
        ],
        out_shape=[
            jax.ShapeDtypeStruct((N, D), f32),
            jax.ShapeDtypeStruct((8, D), f32),
        ],
    )(acc2[:N], acc2[NP:NP + N], den2[:N], den2[NP:NP + N], bias_flat,
      W_fc.T, b_fc.reshape(1, D), ln_gamma.reshape(1, D), ln_beta.reshape(1, D))

    out = pl.pallas_call(
        _tc_fin,
        grid=(G,),
        in_specs=[
            pl.BlockSpec((8, D), lambda i: (0, 0)),
            pl.BlockSpec((D, D), lambda i: (0, 0)),
            pl.BlockSpec((1, D), lambda i: (0, 0)),
            pl.BlockSpec((B, D), lambda i: (i, 0)),
        ],
        out_specs=pl.BlockSpec((B, D), lambda i: (i, 0)),
        out_shape=jax.ShapeDtypeStruct((N, D), f32),
    )(xs, W_g.T, b_g.reshape(1, D), xl)

    return out
